# baseline (speedup 1.0000x reference)
import functools

import jax
import jax.numpy as jnp
from jax.experimental import pallas as pl
from jax.experimental.pallas import tpu as pltpu

_BT = 400
_SLOTS = 3
_LOOKAHEAD = 2


def _tile_copy(adj_hbm, buf, sem, t, *, bt, slots):
    slot = jax.lax.rem(t, slots)
    return pltpu.make_async_copy(
        adj_hbm.at[pl.ds(t * bt, bt), :],
        buf.at[slot],
        sem.at[slot],
    )


def _fused_body(adj_hbm, feat_ref, ws_ref, wn_ref, out_ref, buf, sem,
                *, bt, nt, slots, lookahead):
    for s in range(min(lookahead, nt)):
        _tile_copy(adj_hbm, buf, sem, s, bt=bt, slots=slots).start()

    def step(t, carry):
        @pl.when(t + lookahead < nt)
        def _prefetch():
            _tile_copy(adj_hbm, buf, sem, t + lookahead,
                       bt=bt, slots=slots).start()

        _tile_copy(adj_hbm, buf, sem, t, bt=bt, slots=slots).wait()
        slot = jax.lax.rem(t, slots)
        neigh = jnp.dot(buf[slot], feat_ref[...],
                        preferred_element_type=jnp.float32)
        neigh = jnp.dot(neigh, wn_ref[...],
                        preferred_element_type=jnp.float32)
        feati = feat_ref[pl.ds(t * bt, bt), :]
        selfp = jnp.dot(feati, ws_ref[...],
                        preferred_element_type=jnp.float32)
        out_ref[pl.ds(t * bt, bt), :] = jnp.maximum(neigh + selfp, 0.0)
        return carry

    jax.lax.fori_loop(0, nt, step, 0)


def kernel(feat, adj, weight_self, weight_neigh):
    n, d_in = feat.shape
    d_out = weight_self.shape[1]
    bt = min(_BT, n)
    nt = n // bt
    return pl.pallas_call(
        functools.partial(_fused_body, bt=bt, nt=nt, slots=_SLOTS,
                          lookahead=min(_LOOKAHEAD, _SLOTS - 1)),
        in_specs=[
            pl.BlockSpec(memory_space=pltpu.MemorySpace.HBM),
            pl.BlockSpec(memory_space=pltpu.MemorySpace.VMEM),
            pl.BlockSpec(memory_space=pltpu.MemorySpace.VMEM),
            pl.BlockSpec(memory_space=pltpu.MemorySpace.VMEM),
        ],
        out_specs=pl.BlockSpec(memory_space=pltpu.MemorySpace.VMEM),
        out_shape=jax.ShapeDtypeStruct((n, d_out), jnp.float32),
        scratch_shapes=[
            pltpu.VMEM((_SLOTS, bt, n), jnp.float32),
            pltpu.SemaphoreType.DMA((_SLOTS,)),
        ],
    )(adj, feat, weight_self, weight_neigh)


# speedup vs baseline: 1.0437x; 1.0437x over previous
import functools

import jax
import jax.numpy as jnp
from jax.experimental import pallas as pl
from jax.experimental.pallas import tpu as pltpu

_BI = 400


def _fused_body(adj_ref, feat_ref, ws_ref, wn_ref, out_ref, *, bi):
    i = pl.program_id(0)
    neigh = jnp.dot(adj_ref[...], feat_ref[...],
                    preferred_element_type=jnp.float32)
    neigh = jnp.dot(neigh, wn_ref[...], preferred_element_type=jnp.float32)
    feati = feat_ref[pl.ds(i * bi, bi), :]
    selfp = jnp.dot(feati, ws_ref[...], preferred_element_type=jnp.float32)
    out_ref[...] = jnp.maximum(neigh + selfp, 0.0)


def kernel(feat, adj, weight_self, weight_neigh):
    n, d_in = feat.shape
    d_out = weight_self.shape[1]
    bi = min(_BI, n)
    ni = n // bi
    return pl.pallas_call(
        functools.partial(_fused_body, bi=bi),
        grid=(ni,),
        in_specs=[
            pl.BlockSpec((bi, n), lambda i: (i, 0)),
            pl.BlockSpec((n, d_in), lambda i: (0, 0)),
            pl.BlockSpec((d_in, d_out), lambda i: (0, 0)),
            pl.BlockSpec((d_in, d_out), lambda i: (0, 0)),
        ],
        out_specs=pl.BlockSpec((bi, d_out), lambda i: (i, 0)),
        out_shape=jax.ShapeDtypeStruct((n, d_out), jnp.float32),
        compiler_params=pltpu.CompilerParams(
            dimension_semantics=("parallel",),
        ),
    )(adj, feat, weight_self, weight_neigh)
